# baseline (speedup 1.0000x reference)
import jax
import jax.numpy as jnp
from jax.experimental import pallas as pl
from jax.experimental.pallas import tpu as pltpu

_EPS = 1e-6
_BT = 128


def _body(a_ref, r_ref, x_ref, w_ref, res_out_ref, q_ref):
    a = a_ref[0]
    r = r_ref[...] + x_ref[...].astype(jnp.float32) * a
    res_out_ref[...] = r
    var = jnp.mean(r * r, axis=-1, keepdims=True)
    y = r * jax.lax.rsqrt(var + _EPS) * w_ref[...]
    q_ref[...] = jnp.clip(jnp.round(y), -128, 127).astype(jnp.int8)


def kernel(residual, x, weight, a, *, interpret=False):
    tokens, hidden = residual.shape
    grid = (tokens // _BT,)
    a_arr = jnp.asarray(a, jnp.float32).reshape(1)
    w2d = weight.reshape(1, hidden)
    res_new, out_i8 = pl.pallas_call(
        _body,
        grid=grid,
        in_specs=[
            pl.BlockSpec(memory_space=pltpu.SMEM),
            pl.BlockSpec((_BT, hidden), lambda i: (i, 0)),
            pl.BlockSpec((_BT, hidden), lambda i: (i, 0)),
            pl.BlockSpec((1, hidden), lambda i: (0, 0)),
        ],
        out_specs=[
            pl.BlockSpec((_BT, hidden), lambda i: (i, 0)),
            pl.BlockSpec((_BT, hidden), lambda i: (i, 0)),
        ],
        out_shape=[
            jax.ShapeDtypeStruct((tokens, hidden), jnp.float32),
            jax.ShapeDtypeStruct((tokens, hidden), jnp.int8),
        ],
        compiler_params=pltpu.CompilerParams(
            dimension_semantics=("arbitrary",),
        ),
        name="dequant_add_rmsnorm_quant",
        interpret=interpret,
    )(a_arr, residual, x, w2d)
    return (res_new, out_i8)


# speedup vs baseline: 1.0461x; 1.0461x over previous
import jax
import jax.numpy as jnp
from jax.experimental import pallas as pl
from jax.experimental.pallas import tpu as pltpu

_EPS = 1e-6
_BT = 512


def _body(a_ref, r_ref, x_ref, w_ref, res_out_ref, q_ref):
    a = a_ref[0]
    r = r_ref[...] + x_ref[...].astype(jnp.float32) * a
    res_out_ref[...] = r
    var = jnp.mean(r * r, axis=-1, keepdims=True)
    y = r * jax.lax.rsqrt(var + _EPS) * w_ref[...]
    q_ref[...] = jnp.clip(jnp.round(y), -128, 127).astype(jnp.int8)


def kernel(residual, x, weight, a, *, interpret=False):
    tokens, hidden = residual.shape
    grid = (tokens // _BT,)
    a_arr = jnp.asarray(a, jnp.float32).reshape(1)
    w2d = weight.reshape(1, hidden)
    res_new, out_i8 = pl.pallas_call(
        _body,
        grid=grid,
        in_specs=[
            pl.BlockSpec(memory_space=pltpu.SMEM),
            pl.BlockSpec((_BT, hidden), lambda i: (i, 0)),
            pl.BlockSpec((_BT, hidden), lambda i: (i, 0)),
            pl.BlockSpec((1, hidden), lambda i: (0, 0)),
        ],
        out_specs=[
            pl.BlockSpec((_BT, hidden), lambda i: (i, 0)),
            pl.BlockSpec((_BT, hidden), lambda i: (i, 0)),
        ],
        out_shape=[
            jax.ShapeDtypeStruct((tokens, hidden), jnp.float32),
            jax.ShapeDtypeStruct((tokens, hidden), jnp.int8),
        ],
        compiler_params=pltpu.CompilerParams(
            dimension_semantics=("arbitrary",),
            vmem_limit_bytes=62 * 1024 * 1024,
        ),
        name="dequant_add_rmsnorm_quant",
        interpret=interpret,
    )(a_arr, residual, x, w2d)
    return (res_new, out_i8)
